# baseline (speedup 1.0000x reference)
import functools

import jax
import jax.numpy as jnp
from jax import lax
from jax.experimental import pallas as pl
from jax.experimental.pallas import tpu as pltpu
from jax.experimental.pallas import tpu_sc as plsc

B = 16384
L = 200
LP = 208
EMB = 16
HID = 200
NCLS = 1000

NSLICE = 4
BS = B // NSLICE

NC = 2
NS = 16
NW = NC * NS
ROWS_PER_W = BS // NW
RB = 8
CHUNK = RB * LP
GW = 128
NG = CHUNK // GW
N_CHUNKS = ROWS_PER_W // RB


def _sc_pool(x_flat, emb):
    mesh = plsc.VectorSubcoreMesh(core_axis_name="c", subcore_axis_name="s")

    @functools.partial(
        pl.kernel,
        mesh=mesh,
        compiler_params=pltpu.CompilerParams(use_tc_tiling_on_sc=False),
        out_type=jax.ShapeDtypeStruct((BS, EMB), jnp.float32),
        scratch_types=[
            pltpu.VMEM((2, CHUNK), jnp.int32),
            pltpu.VMEM((2 * CHUNK, EMB), jnp.float32),
            pltpu.VMEM((ROWS_PER_W, EMB), jnp.float32),
            pltpu.SemaphoreType.DMA,
            pltpu.SemaphoreType.DMA,
        ],
    )
    def k(x_hbm, emb_hbm, out_hbm, idx_v, rows_v, acc_v, sem0, sem1):
        wid = lax.axis_index("s") * NC + lax.axis_index("c")
        base_row = wid * ROWS_PER_W
        sems = (sem0, sem1)

        def fire(chunk, buf):
            row0 = base_row + chunk * RB
            pltpu.sync_copy(x_hbm.at[pl.ds(row0 * LP, CHUNK)], idx_v.at[buf])
            for g in range(NG):
                pltpu.async_copy(
                    emb_hbm.at[idx_v.at[buf, pl.ds(g * GW, GW)]],
                    rows_v.at[pl.ds(buf * CHUNK + g * GW, GW)],
                    sems[buf],
                )

        def drain(buf):
            for g in range(NG):
                pltpu.make_async_copy(
                    emb_hbm.at[idx_v.at[buf, pl.ds(g * GW, GW)]],
                    rows_v.at[pl.ds(buf * CHUNK + g * GW, GW)],
                    sems[buf],
                ).wait()

        def accum(chunk, buf):
            for r in range(RB):
                base = buf * CHUNK + r * LP

                def body(j, accs):
                    a0, a1, a2, a3 = accs
                    o = base + j * 4
                    return (
                        a0 + rows_v[o, :],
                        a1 + rows_v[o + 1, :],
                        a2 + rows_v[o + 2, :],
                        a3 + rows_v[o + 3, :],
                    )

                z = jnp.zeros((EMB,), jnp.float32)
                a0, a1, a2, a3 = lax.fori_loop(0, LP // 4, body, (z, z, z, z))
                acc_v[chunk * RB + r, :] = (a0 + a1) + (a2 + a3)

        fire(0, 0)

        @pl.loop(0, N_CHUNKS, step=2)
        def _(c):
            fire(c + 1, 1)
            drain(0)
            accum(c, 0)
            fire(jnp.minimum(c + 2, N_CHUNKS - 2), 0)
            drain(1)
            accum(c + 1, 1)

        drain(0)
        pltpu.sync_copy(acc_v, out_hbm.at[pl.ds(base_row, ROWS_PER_W)])

    return k(x_flat, emb)


BB = 256


def _tc_head(sums, xp, W1, b1, W2, b2):
    def body(sums_ref, x_ref, w1_ref, b1_ref, w2_ref, b2_ref, out_ref):
        xi = x_ref[...]
        nz = jnp.sum((xi != 0).astype(jnp.float32), axis=1, keepdims=True)
        e = sums_ref[...] / nz
        norm = jnp.sqrt(jnp.sum(e * e, axis=1, keepdims=True))
        e = e / jnp.maximum(norm, 1e-12)
        h = lax.dot_general(
            e, w1_ref[...], (((1,), (1,)), ((), ())),
            preferred_element_type=jnp.float32,
        ) + b1_ref[...]
        h = jnp.maximum(h, 0.0)
        logits = lax.dot_general(
            w2_ref[...], h, (((1,), (1,)), ((), ())),
            preferred_element_type=jnp.float32,
        ) + b2_ref[...]
        m = jnp.max(logits, axis=0, keepdims=True)
        ex = jnp.exp(logits - m)
        out_ref[...] = ex / jnp.sum(ex, axis=0, keepdims=True)

    return pl.pallas_call(
        body,
        grid=(BS // BB,),
        in_specs=[
            pl.BlockSpec((BB, EMB), lambda i: (i, 0)),
            pl.BlockSpec((BB, LP), lambda i: (i, 0)),
            pl.BlockSpec((HID, EMB), lambda i: (0, 0)),
            pl.BlockSpec((1, HID), lambda i: (0, 0)),
            pl.BlockSpec((NCLS, HID), lambda i: (0, 0)),
            pl.BlockSpec((NCLS, 1), lambda i: (0, 0)),
        ],
        out_specs=pl.BlockSpec((NCLS, BB), lambda i: (0, i)),
        out_shape=jax.ShapeDtypeStruct((NCLS, BS), jnp.float32),
    )(sums, xp, W1, b1.reshape(1, HID), W2, b2.reshape(NCLS, 1))


def kernel(x, emb, W1, b1, W2, b2):
    xp = jnp.pad(x, ((0, 0), (0, LP - L)))
    x_flat = xp.reshape(-1)
    parts = []
    for s in range(NSLICE):
        sums = _sc_pool(
            lax.slice(x_flat, (s * BS * LP,), ((s + 1) * BS * LP,)), emb
        )
        xs = lax.slice(xp, (s * BS, 0), ((s + 1) * BS, LP))
        parts.append(_tc_head(sums, xs, W1, b1, W2, b2))
    out_t = jnp.concatenate(parts, axis=1)
    return out_t.T


# speedup vs baseline: 1.0867x; 1.0867x over previous
import functools

import jax
import jax.numpy as jnp
from jax import lax
from jax.experimental import pallas as pl
from jax.experimental.pallas import tpu as pltpu
from jax.experimental.pallas import tpu_sc as plsc

B = 16384
L = 200
LP = 208
EMB = 16
HID = 200
NCLS = 1000

NC = 2
NS = 16
NW = NC * NS
ROWS_PER_W = B // NW
RB = 8
CHUNK = RB * LP
GW = 128
NG = CHUNK // GW
N_CHUNKS = ROWS_PER_W // RB


def _sc_pool(x_flat, emb):
    mesh = plsc.VectorSubcoreMesh(core_axis_name="c", subcore_axis_name="s")

    @functools.partial(
        pl.kernel,
        mesh=mesh,
        compiler_params=pltpu.CompilerParams(use_tc_tiling_on_sc=False),
        out_type=jax.ShapeDtypeStruct((B, EMB), jnp.float32),
        scratch_types=[
            pltpu.VMEM((2, CHUNK), jnp.int32),
            pltpu.VMEM((2 * CHUNK, EMB), jnp.float32),
            pltpu.VMEM((ROWS_PER_W, EMB), jnp.float32),
            pltpu.SemaphoreType.DMA,
            pltpu.SemaphoreType.DMA,
        ],
    )
    def k(x_hbm, emb_hbm, out_hbm, idx_v, rows_v, acc_v, sem0, sem1):
        wid = lax.axis_index("s") * NC + lax.axis_index("c")
        base_row = wid * ROWS_PER_W
        sems = (sem0, sem1)

        def fire(chunk, buf):
            row0 = base_row + chunk * RB
            pltpu.sync_copy(x_hbm.at[pl.ds(row0 * LP, CHUNK)], idx_v.at[buf])
            for g in range(NG):
                pltpu.async_copy(
                    emb_hbm.at[idx_v.at[buf, pl.ds(g * GW, GW)]],
                    rows_v.at[pl.ds(buf * CHUNK + g * GW, GW)],
                    sems[buf],
                )

        def drain(buf):
            for g in range(NG):
                pltpu.make_async_copy(
                    emb_hbm.at[idx_v.at[buf, pl.ds(g * GW, GW)]],
                    rows_v.at[pl.ds(buf * CHUNK + g * GW, GW)],
                    sems[buf],
                ).wait()

        def accum(chunk, buf):
            for r in range(RB):
                base = buf * CHUNK + r * LP

                def body(j, accs):
                    a0, a1, a2, a3 = accs
                    o = base + j * 4
                    return (
                        a0 + rows_v[o, :],
                        a1 + rows_v[o + 1, :],
                        a2 + rows_v[o + 2, :],
                        a3 + rows_v[o + 3, :],
                    )

                z = jnp.zeros((EMB,), jnp.float32)
                a0, a1, a2, a3 = lax.fori_loop(0, LP // 4, body, (z, z, z, z))
                acc_v[chunk * RB + r, :] = (a0 + a1) + (a2 + a3)

        fire(0, 0)

        @pl.loop(0, N_CHUNKS, step=2)
        def _(c):
            fire(c + 1, 1)
            drain(0)
            accum(c, 0)
            fire(jnp.minimum(c + 2, N_CHUNKS - 2), 0)
            drain(1)
            accum(c + 1, 1)

        drain(0)
        pltpu.sync_copy(acc_v, out_hbm.at[pl.ds(base_row, ROWS_PER_W)])

    return k(x_flat, emb)


BB = 256


def _tc_head(sums, xp, W1, b1, W2, b2):
    def body(sums_ref, x_ref, w1_ref, b1_ref, w2_ref, b2_ref, out_ref):
        xi = x_ref[...]
        nz = jnp.sum((xi != 0).astype(jnp.float32), axis=1, keepdims=True)
        e = sums_ref[...] / nz
        norm = jnp.sqrt(jnp.sum(e * e, axis=1, keepdims=True))
        e = e / jnp.maximum(norm, 1e-12)
        h = lax.dot_general(
            e, w1_ref[...], (((1,), (1,)), ((), ())),
            preferred_element_type=jnp.float32,
        ) + b1_ref[...]
        h = jnp.maximum(h, 0.0)
        logits = lax.dot_general(
            w2_ref[...].astype(jnp.bfloat16),
            h.astype(jnp.bfloat16),
            (((1,), (1,)), ((), ())),
            preferred_element_type=jnp.float32,
        ) + b2_ref[...]
        m = jnp.max(logits, axis=0, keepdims=True)
        ex = jnp.exp(logits - m)
        out_ref[...] = ex / jnp.sum(ex, axis=0, keepdims=True)

    return pl.pallas_call(
        body,
        grid=(B // BB,),
        in_specs=[
            pl.BlockSpec((BB, EMB), lambda i: (i, 0)),
            pl.BlockSpec((BB, LP), lambda i: (i, 0)),
            pl.BlockSpec((HID, EMB), lambda i: (0, 0)),
            pl.BlockSpec((1, HID), lambda i: (0, 0)),
            pl.BlockSpec((NCLS, HID), lambda i: (0, 0)),
            pl.BlockSpec((NCLS, 1), lambda i: (0, 0)),
        ],
        out_specs=pl.BlockSpec((NCLS, BB), lambda i: (0, i)),
        out_shape=jax.ShapeDtypeStruct((NCLS, B), jnp.float32),
    )(sums, xp, W1, b1.reshape(1, HID), W2, b2.reshape(NCLS, 1))


def kernel(x, emb, W1, b1, W2, b2):
    xp = jnp.pad(x, ((0, 0), (0, LP - L)))
    x_flat = xp.reshape(-1)
    sums = _sc_pool(x_flat, emb)
    return _tc_head(sums, xp, W1, b1, W2, b2).T


# speedup vs baseline: 1.1955x; 1.1001x over previous
import functools

import jax
import jax.numpy as jnp
from jax import lax
from jax.experimental import pallas as pl
from jax.experimental.pallas import tpu as pltpu
from jax.experimental.pallas import tpu_sc as plsc

B = 16384
L = 200
LP = 208
EMB = 16
HID = 200
NCLS = 1000

NUM_VOCAB_P1 = 1000001

NC = 2
NS = 16
NW = NC * NS
ROWS_PER_W = B // NW
RB = 8
CHUNK = RB * LP
GW = 128
NG = CHUNK // GW
N_CHUNKS = ROWS_PER_W // RB


VC = 8192
NVB = 123
V_PAD = NVB * VC


def _linearize_table(emb):
    def body(in_ref, out_ref):
        eye = jnp.eye(EMB, dtype=jnp.float32)
        t = lax.dot_general(
            in_ref[...], eye, (((0,), (0,)), ((), ())),
            preferred_element_type=jnp.float32,
        ).reshape(VC // 8, 8, EMB)
        out_ref[...] = jnp.concatenate(
            [t[:, k, :] for k in range(8)], axis=1
        )

    out = pl.pallas_call(
        body,
        grid=(NVB,),
        in_specs=[pl.BlockSpec((EMB, VC), lambda i: (0, i))],
        out_specs=pl.BlockSpec((VC // 8, 128), lambda i: (i, 0)),
        out_shape=jax.ShapeDtypeStruct((V_PAD // 8, 128), jnp.float32),
    )(emb.T)
    return out.reshape(V_PAD, EMB)


def _sc_pool(x_flat, emb):
    mesh = plsc.VectorSubcoreMesh(core_axis_name="c", subcore_axis_name="s")

    @functools.partial(
        pl.kernel,
        mesh=mesh,
        compiler_params=pltpu.CompilerParams(use_tc_tiling_on_sc=False),
        out_type=jax.ShapeDtypeStruct((B, EMB), jnp.float32),
        scratch_types=[
            pltpu.VMEM((2, CHUNK), jnp.int32),
            pltpu.VMEM((2 * CHUNK, EMB), jnp.float32),
            pltpu.VMEM((ROWS_PER_W, EMB), jnp.float32),
            pltpu.SemaphoreType.DMA,
            pltpu.SemaphoreType.DMA,
        ],
    )
    def k(x_hbm, emb_hbm, out_hbm, idx_v, rows_v, acc_v, sem0, sem1):
        wid = lax.axis_index("s") * NC + lax.axis_index("c")
        base_row = wid * ROWS_PER_W
        sems = (sem0, sem1)

        def fire(chunk, buf):
            row0 = base_row + chunk * RB
            pltpu.sync_copy(x_hbm.at[pl.ds(row0 * LP, CHUNK)], idx_v.at[buf])
            for g in range(NG):
                pltpu.async_copy(
                    emb_hbm.at[idx_v.at[buf, pl.ds(g * GW, GW)]],
                    rows_v.at[pl.ds(buf * CHUNK + g * GW, GW)],
                    sems[buf],
                )

        def drain(buf):
            for g in range(NG):
                pltpu.make_async_copy(
                    emb_hbm.at[idx_v.at[buf, pl.ds(g * GW, GW)]],
                    rows_v.at[pl.ds(buf * CHUNK + g * GW, GW)],
                    sems[buf],
                ).wait()

        def accum(chunk, buf):
            for r in range(RB):
                base = buf * CHUNK + r * LP

                def body(j, accs):
                    a0, a1, a2, a3 = accs
                    o = base + j * 4
                    return (
                        a0 + rows_v[o, :],
                        a1 + rows_v[o + 1, :],
                        a2 + rows_v[o + 2, :],
                        a3 + rows_v[o + 3, :],
                    )

                z = jnp.zeros((EMB,), jnp.float32)
                a0, a1, a2, a3 = lax.fori_loop(0, LP // 4, body, (z, z, z, z))
                acc_v[chunk * RB + r, :] = (a0 + a1) + (a2 + a3)

        fire(0, 0)

        @pl.loop(0, N_CHUNKS, step=2)
        def _(c):
            fire(c + 1, 1)
            drain(0)
            accum(c, 0)
            fire(jnp.minimum(c + 2, N_CHUNKS - 2), 0)
            drain(1)
            accum(c + 1, 1)

        drain(0)
        pltpu.sync_copy(acc_v, out_hbm.at[pl.ds(base_row, ROWS_PER_W)])

    return k(x_flat, emb)


BB = 256


def _tc_head(sums, xp, W1, b1, W2, b2):
    def body(sums_ref, x_ref, w1_ref, b1_ref, w2_ref, b2_ref, out_ref):
        xi = x_ref[...]
        nz = jnp.sum((xi != 0).astype(jnp.float32), axis=1, keepdims=True)
        e = sums_ref[...] / nz
        norm = jnp.sqrt(jnp.sum(e * e, axis=1, keepdims=True))
        e = e / jnp.maximum(norm, 1e-12)
        h = lax.dot_general(
            e, w1_ref[...], (((1,), (1,)), ((), ())),
            preferred_element_type=jnp.float32,
        ) + b1_ref[...]
        h = jnp.maximum(h, 0.0)
        logits = lax.dot_general(
            w2_ref[...].astype(jnp.bfloat16),
            h.astype(jnp.bfloat16),
            (((1,), (1,)), ((), ())),
            preferred_element_type=jnp.float32,
        ) + b2_ref[...]
        m = jnp.max(logits, axis=0, keepdims=True)
        ex = jnp.exp(logits - m)
        out_ref[...] = ex / jnp.sum(ex, axis=0, keepdims=True)

    return pl.pallas_call(
        body,
        grid=(B // BB,),
        in_specs=[
            pl.BlockSpec((BB, EMB), lambda i: (i, 0)),
            pl.BlockSpec((BB, LP), lambda i: (i, 0)),
            pl.BlockSpec((HID, EMB), lambda i: (0, 0)),
            pl.BlockSpec((1, HID), lambda i: (0, 0)),
            pl.BlockSpec((NCLS, HID), lambda i: (0, 0)),
            pl.BlockSpec((NCLS, 1), lambda i: (0, 0)),
        ],
        out_specs=pl.BlockSpec((NCLS, BB), lambda i: (0, i)),
        out_shape=jax.ShapeDtypeStruct((NCLS, B), jnp.float32),
    )(sums, xp, W1, b1.reshape(1, HID), W2, b2.reshape(NCLS, 1))


def kernel(x, emb, W1, b1, W2, b2):
    xp = jnp.pad(x, ((0, 0), (0, LP - L)))
    x_flat = xp.reshape(-1)
    sums = _sc_pool(x_flat, _linearize_table(emb))
    return _tc_head(sums, xp, W1, b1, W2, b2).T


# speedup vs baseline: 1.2312x; 1.0299x over previous
import functools

import jax
import jax.numpy as jnp
from jax import lax
from jax.experimental import pallas as pl
from jax.experimental.pallas import tpu as pltpu
from jax.experimental.pallas import tpu_sc as plsc

B = 16384
L = 200
LP = 208
EMB = 16
HID = 200
NCLS = 1000

NUM_VOCAB_P1 = 1000001

NC = 2
NS = 16
NW = NC * NS
ROWS_PER_W = B // NW
RB = 8
CHUNK = RB * LP
GW = 128
NG = CHUNK // GW
N_CHUNKS = ROWS_PER_W // RB


VC = 8192
NVB = 123
V_PAD = NVB * VC


def _linearize_table(emb):
    def body(in_ref, out_ref):
        eye = jnp.eye(EMB, dtype=jnp.float32)
        t = lax.dot_general(
            in_ref[...], eye, (((0,), (0,)), ((), ())),
            preferred_element_type=jnp.float32,
        ).reshape(VC // 8, 8, EMB)
        for k in range(8):
            out_ref[:, k * EMB:(k + 1) * EMB] = t[:, k, :]

    out = pl.pallas_call(
        body,
        grid=(NVB,),
        in_specs=[pl.BlockSpec((EMB, VC), lambda i: (0, i))],
        out_specs=pl.BlockSpec((VC // 8, 128), lambda i: (i, 0)),
        out_shape=jax.ShapeDtypeStruct((V_PAD // 8, 128), jnp.float32),
    )(emb.T)
    return out.reshape(V_PAD, EMB)


def _sc_pool(x_flat, emb):
    mesh = plsc.VectorSubcoreMesh(core_axis_name="c", subcore_axis_name="s")

    @functools.partial(
        pl.kernel,
        mesh=mesh,
        compiler_params=pltpu.CompilerParams(use_tc_tiling_on_sc=False),
        out_type=jax.ShapeDtypeStruct((B, EMB), jnp.float32),
        scratch_types=[
            pltpu.VMEM((2, CHUNK), jnp.int32),
            pltpu.VMEM((2 * CHUNK, EMB), jnp.float32),
            pltpu.VMEM((ROWS_PER_W, EMB), jnp.float32),
            pltpu.SemaphoreType.DMA,
            pltpu.SemaphoreType.DMA,
        ],
    )
    def k(x_hbm, emb_hbm, out_hbm, idx_v, rows_v, acc_v, sem0, sem1):
        wid = lax.axis_index("s") * NC + lax.axis_index("c")
        base_row = wid * ROWS_PER_W
        sems = (sem0, sem1)

        def fire(chunk, buf):
            row0 = base_row + chunk * RB
            pltpu.sync_copy(x_hbm.at[pl.ds(row0 * LP, CHUNK)], idx_v.at[buf])
            for g in range(NG):
                pltpu.async_copy(
                    emb_hbm.at[idx_v.at[buf, pl.ds(g * GW, GW)]],
                    rows_v.at[pl.ds(buf * CHUNK + g * GW, GW)],
                    sems[buf],
                )

        def drain(buf):
            for g in range(NG):
                pltpu.make_async_copy(
                    emb_hbm.at[idx_v.at[buf, pl.ds(g * GW, GW)]],
                    rows_v.at[pl.ds(buf * CHUNK + g * GW, GW)],
                    sems[buf],
                ).wait()

        def accum(chunk, buf):
            for r in range(RB):
                base = buf * CHUNK + r * LP

                def body(j, accs):
                    a0, a1, a2, a3 = accs
                    o = base + j * 4
                    return (
                        a0 + rows_v[o, :],
                        a1 + rows_v[o + 1, :],
                        a2 + rows_v[o + 2, :],
                        a3 + rows_v[o + 3, :],
                    )

                z = jnp.zeros((EMB,), jnp.float32)
                a0, a1, a2, a3 = lax.fori_loop(0, LP // 4, body, (z, z, z, z))
                acc_v[chunk * RB + r, :] = (a0 + a1) + (a2 + a3)

        fire(0, 0)

        @pl.loop(0, N_CHUNKS, step=2)
        def _(c):
            fire(c + 1, 1)
            drain(0)
            accum(c, 0)
            fire(jnp.minimum(c + 2, N_CHUNKS - 2), 0)
            drain(1)
            accum(c + 1, 1)

        drain(0)
        pltpu.sync_copy(acc_v, out_hbm.at[pl.ds(base_row, ROWS_PER_W)])

    return k(x_flat, emb)


BB = 256


def _tc_head(sums, xp, W1, b1, W2, b2):
    def body(sums_ref, x_ref, w1_ref, b1_ref, w2_ref, b2_ref, out_ref):
        xi = x_ref[...]
        nz = jnp.sum((xi != 0).astype(jnp.float32), axis=1, keepdims=True)
        e = sums_ref[...] / nz
        norm = jnp.sqrt(jnp.sum(e * e, axis=1, keepdims=True))
        e = e / jnp.maximum(norm, 1e-12)
        h = lax.dot_general(
            e, w1_ref[...], (((1,), (1,)), ((), ())),
            preferred_element_type=jnp.float32,
        ) + b1_ref[...]
        h = jnp.maximum(h, 0.0)
        logits = lax.dot_general(
            w2_ref[...].astype(jnp.bfloat16),
            h.astype(jnp.bfloat16),
            (((1,), (1,)), ((), ())),
            preferred_element_type=jnp.float32,
        ) + b2_ref[...]
        m = jnp.max(logits, axis=0, keepdims=True)
        ex = jnp.exp(logits - m)
        out_ref[...] = ex / jnp.sum(ex, axis=0, keepdims=True)

    return pl.pallas_call(
        body,
        grid=(B // BB,),
        in_specs=[
            pl.BlockSpec((BB, EMB), lambda i: (i, 0)),
            pl.BlockSpec((BB, LP), lambda i: (i, 0)),
            pl.BlockSpec((HID, EMB), lambda i: (0, 0)),
            pl.BlockSpec((1, HID), lambda i: (0, 0)),
            pl.BlockSpec((NCLS, HID), lambda i: (0, 0)),
            pl.BlockSpec((NCLS, 1), lambda i: (0, 0)),
        ],
        out_specs=pl.BlockSpec((NCLS, BB), lambda i: (0, i)),
        out_shape=jax.ShapeDtypeStruct((NCLS, B), jnp.float32),
    )(sums, xp, W1, b1.reshape(1, HID), W2, b2.reshape(NCLS, 1))


def kernel(x, emb, W1, b1, W2, b2):
    xp = jnp.pad(x, ((0, 0), (0, LP - L)))
    x_flat = xp.reshape(-1)
    sums = _sc_pool(x_flat, _linearize_table(emb))
    return _tc_head(sums, xp, W1, b1, W2, b2).T


# speedup vs baseline: 1.2544x; 1.0188x over previous
import functools

import jax
import jax.numpy as jnp
from jax import lax
from jax.experimental import pallas as pl
from jax.experimental.pallas import tpu as pltpu
from jax.experimental.pallas import tpu_sc as plsc

B = 16384
L = 200
LP = 208
EMB = 16
HID = 200
NCLS = 1000

NUM_VOCAB_P1 = 1000001

NC = 2
NS = 16
NW = NC * NS
ROWS_PER_W = B // NW
RB = 8
CHUNK = RB * LP
GW = 128
NG = CHUNK // GW
N_CHUNKS = ROWS_PER_W // RB


VC = 8192
NVB = 123
V_PAD = NVB * VC


def _linearize_table(emb):
    def body(in_ref, out_ref):
        eye = jnp.eye(EMB, dtype=jnp.float32)
        t = lax.dot_general(
            in_ref[...], eye, (((0,), (0,)), ((), ())),
            preferred_element_type=jnp.float32,
        ).reshape(VC // 8, 8, EMB)
        for k in range(8):
            out_ref[:, k * EMB:(k + 1) * EMB] = t[:, k, :]

    out = pl.pallas_call(
        body,
        grid=(NVB,),
        in_specs=[pl.BlockSpec((EMB, VC), lambda i: (0, i))],
        out_specs=pl.BlockSpec((VC // 8, 128), lambda i: (i, 0)),
        out_shape=jax.ShapeDtypeStruct((V_PAD // 8, 128), jnp.float32),
    )(emb.T)
    return out.reshape(V_PAD, EMB)


def _sc_pool(x_flat, emb):
    mesh = plsc.VectorSubcoreMesh(core_axis_name="c", subcore_axis_name="s")

    @functools.partial(
        pl.kernel,
        mesh=mesh,
        compiler_params=pltpu.CompilerParams(use_tc_tiling_on_sc=False),
        out_type=jax.ShapeDtypeStruct((B, EMB), jnp.float32),
        scratch_types=[
            pltpu.VMEM((2, CHUNK), jnp.int32),
            pltpu.VMEM((2 * CHUNK, EMB), jnp.float32),
            pltpu.VMEM((ROWS_PER_W, EMB), jnp.float32),
            pltpu.SemaphoreType.DMA,
            pltpu.SemaphoreType.DMA,
        ],
    )
    def k(x_hbm, emb_hbm, out_hbm, idx_v, rows_v, acc_v, sem0, sem1):
        wid = lax.axis_index("s") * NC + lax.axis_index("c")
        base_row = wid * ROWS_PER_W
        sems = (sem0, sem1)

        def fire(chunk, buf):
            row0 = base_row + chunk * RB
            pltpu.sync_copy(x_hbm.at[pl.ds(row0 * LP, CHUNK)], idx_v.at[buf])
            for g in range(NG):
                pltpu.async_copy(
                    emb_hbm.at[idx_v.at[buf, pl.ds(g * GW, GW)]],
                    rows_v.at[pl.ds(buf * CHUNK + g * GW, GW)],
                    sems[buf],
                )

        def drain(buf):
            for g in range(NG):
                pltpu.make_async_copy(
                    emb_hbm.at[idx_v.at[buf, pl.ds(g * GW, GW)]],
                    rows_v.at[pl.ds(buf * CHUNK + g * GW, GW)],
                    sems[buf],
                ).wait()

        def accum(chunk, buf):
            for r in range(RB):
                base = buf * CHUNK + r * LP

                def body(j, accs):
                    a0, a1, a2, a3 = accs
                    o = base + j * 4
                    return (
                        a0 + rows_v[o, :],
                        a1 + rows_v[o + 1, :],
                        a2 + rows_v[o + 2, :],
                        a3 + rows_v[o + 3, :],
                    )

                z = jnp.zeros((EMB,), jnp.float32)
                a0, a1, a2, a3 = lax.fori_loop(0, LP // 4, body, (z, z, z, z))
                acc_v[chunk * RB + r, :] = (a0 + a1) + (a2 + a3)

        fire(0, 0)

        @pl.loop(0, N_CHUNKS, step=2)
        def _(c):
            fire(c + 1, 1)
            drain(0)
            accum(c, 0)
            fire(jnp.minimum(c + 2, N_CHUNKS - 2), 0)
            drain(1)
            accum(c + 1, 1)

        drain(0)
        pltpu.sync_copy(acc_v, out_hbm.at[pl.ds(base_row, ROWS_PER_W)])

    return k(x_flat, emb)


BB = 512


def _tc_head(sums, xp, W1, b1, W2, b2):
    def body(sums_ref, x_ref, w1_ref, b1_ref, w2_ref, b2_ref, out_ref):
        xi = x_ref[...]
        nz = jnp.sum((xi != 0).astype(jnp.float32), axis=1, keepdims=True)
        e = sums_ref[...] / nz
        norm = jnp.sqrt(jnp.sum(e * e, axis=1, keepdims=True))
        e = e / jnp.maximum(norm, 1e-12)
        h = lax.dot_general(
            e, w1_ref[...], (((1,), (1,)), ((), ())),
            preferred_element_type=jnp.float32,
        ) + b1_ref[...]
        h = jnp.maximum(h, 0.0)
        logits = lax.dot_general(
            w2_ref[...].astype(jnp.bfloat16),
            h.astype(jnp.bfloat16),
            (((1,), (1,)), ((), ())),
            preferred_element_type=jnp.float32,
        ) + b2_ref[...]
        m = jnp.max(logits, axis=0, keepdims=True)
        ex = jnp.exp(logits - m)
        out_ref[...] = ex / jnp.sum(ex, axis=0, keepdims=True)

    return pl.pallas_call(
        body,
        grid=(B // BB,),
        in_specs=[
            pl.BlockSpec((BB, EMB), lambda i: (i, 0)),
            pl.BlockSpec((BB, LP), lambda i: (i, 0)),
            pl.BlockSpec((HID, EMB), lambda i: (0, 0)),
            pl.BlockSpec((1, HID), lambda i: (0, 0)),
            pl.BlockSpec((NCLS, HID), lambda i: (0, 0)),
            pl.BlockSpec((NCLS, 1), lambda i: (0, 0)),
        ],
        out_specs=pl.BlockSpec((NCLS, BB), lambda i: (0, i)),
        out_shape=jax.ShapeDtypeStruct((NCLS, B), jnp.float32),
    )(sums, xp, W1, b1.reshape(1, HID), W2, b2.reshape(NCLS, 1))


def kernel(x, emb, W1, b1, W2, b2):
    xp = jnp.pad(x, ((0, 0), (0, LP - L)))
    x_flat = xp.reshape(-1)
    sums = _sc_pool(x_flat, _linearize_table(emb))
    return _tc_head(sums, xp, W1, b1, W2, b2).T
